# baseline (speedup 1.0000x reference)
import functools

import jax
import jax.numpy as jnp
from jax import lax
from jax.experimental import pallas as pl
from jax.experimental.pallas import tpu as pltpu
from jax.experimental.pallas import tpu_sc as plsc

_LANES = 128
_PAIRW = 128


@functools.lru_cache(maxsize=None)
def _make_gather(seq: int, batch: int, hidden: int):
    info = plsc.get_sparse_core_info()
    nc, ns = info.num_cores, info.num_subcores
    nw = nc * ns
    assert batch == nw * _LANES and seq % 2 == 0

    mesh = plsc.VectorSubcoreMesh(core_axis_name="c", subcore_axis_name="s")

    @functools.partial(
        pl.kernel,
        mesh=mesh,
        out_type=jax.ShapeDtypeStruct((seq, hidden, batch), jnp.float32),
        scratch_types=[
            pltpu.VMEM((seq, _LANES), jnp.int32),
            pltpu.VMEM((_LANES,), jnp.int32),
            pltpu.VMEM((_LANES,), jnp.int32),
            pltpu.VMEM((_LANES,), jnp.int32),
            pltpu.VMEM((_LANES,), jnp.int32),
            pltpu.VMEM((_LANES, _PAIRW), jnp.float32),
            pltpu.VMEM((_LANES, _PAIRW), jnp.float32),
            pltpu.VMEM((hidden, _LANES), jnp.float32),
            pltpu.VMEM((hidden, _LANES), jnp.float32),
            pltpu.SemaphoreType.DMA,
            pltpu.SemaphoreType.DMA,
            pltpu.SemaphoreType.DMA,
            pltpu.SemaphoreType.DMA,
        ],
        compiler_params=pltpu.CompilerParams(
            needs_layout_passes=False, skip_device_barrier=True
        ),
    )
    def k(table_hbm, idx_hbm, out_hbm, idx_s, r0, r1, a0, a1,
          g0, g1, t0, t1, gs0, gs1, ss0, ss1):
        wid = lax.axis_index("s") * nc + lax.axis_index("c")
        b0 = wid * _LANES

        pltpu.sync_copy(idx_hbm.at[:, pl.ds(b0, _LANES)], idx_s)

        rbuf = (r0, r1)
        abuf = (a0, a1)
        gbuf = ((g0, gs0), (g1, gs1))
        tbuf = ((t0, ss0), (t1, ss1))

        def compute_r(s, par):
            r, a = rbuf[par], abuf[par]
            for kk in range(8):
                v = idx_s[s, pl.ds(16 * kk, 16)]
                r[pl.ds(16 * kk, 16)] = lax.shift_right_logical(v, 1)
                a[pl.ds(16 * kk, 16)] = lax.shift_left(
                    lax.bitwise_and(v, 1), 6
                )

        def fire(par):
            g, sem = gbuf[par]
            pltpu.async_copy(table_hbm.at[rbuf[par]], g, sem)

        def wait_gather(par):
            g, sem = gbuf[par]
            pltpu.make_async_copy(table_hbm.at[rbuf[par]], g, sem).wait()

        def wait_store(par):
            t, sem = tbuf[par]
            pltpu.make_async_copy(
                table_hbm.at[rbuf[par].at[pl.ds(0, 64)]],
                t.at[pl.ds(0, 64)],
                sem,
            ).wait()

        rowc = jnp.arange(16, dtype=jnp.int32)

        def transpose(par):
            g = gbuf[par][0]
            t = tbuf[par][0]
            a = abuf[par]

            def bg_body(bg, carry):
                rows = rowc + 16 * bg
                cols0 = a[pl.ds(16 * bg, 16)]
                for h in range(hidden):
                    v = plsc.load_gather(g, [rows, cols0 + h])
                    t[h, pl.ds(16 * bg, 16)] = v
                return carry

            lax.fori_loop(0, 8, bg_body, 0, unroll=False)

        def store(s, par):
            t, sem = tbuf[par]
            pltpu.async_copy(t, out_hbm.at[s, :, pl.ds(b0, _LANES)], sem)

        compute_r(0, 0)
        fire(0)
        compute_r(1, 1)
        fire(1)

        def body(p, carry):
            for par in (0, 1):
                s = 2 * p + par
                wait_gather(par)

                @pl.when(p > 0)
                def _():
                    wait_store(par)

                transpose(par)
                store(s, par)

                @pl.when(s + 2 < seq)
                def _():
                    compute_r(s + 2, par)
                    fire(par)

            return carry

        lax.fori_loop(0, seq // 2, body, 0)
        wait_store(0)
        wait_store(1)

    return k


def kernel(inputs, embed_table):
    b, s = inputs.shape
    v, d = embed_table.shape
    idx_t = inputs.T.astype(jnp.int32)
    tab2 = embed_table.reshape(v // 2, 2 * d)
    out = _make_gather(s, b, d)(tab2, idx_t)
    return out.transpose(2, 0, 1)


# speedup vs baseline: 1.8229x; 1.8229x over previous
import functools

import jax
import jax.numpy as jnp
from jax import lax
from jax.experimental import pallas as pl
from jax.experimental.pallas import tpu as pltpu
from jax.experimental.pallas import tpu_sc as plsc

_LANES = 128
_K = 2
_CHUNK = _K * _LANES
_PADW = 128


@functools.lru_cache(maxsize=None)
def _make_gather(total_rows: int, hidden: int):
    info = plsc.get_sparse_core_info()
    nc, ns = info.num_cores, info.num_subcores
    nw = nc * ns
    rows_per_w = total_rows // nw
    n_chunks = rows_per_w // _CHUNK
    n_pairs = n_chunks // 2
    assert rows_per_w % _CHUNK == 0 and n_chunks % 2 == 0

    mesh = plsc.VectorSubcoreMesh(core_axis_name="c", subcore_axis_name="s")

    @functools.partial(
        pl.kernel,
        mesh=mesh,
        out_type=jax.ShapeDtypeStruct((total_rows, _PADW), jnp.float32),
        scratch_types=[
            pltpu.VMEM((rows_per_w,), jnp.int32),
            pltpu.VMEM((_CHUNK, _PADW), jnp.float32),
            pltpu.VMEM((_CHUNK, _PADW), jnp.float32),
            pltpu.SemaphoreType.DMA,
            pltpu.SemaphoreType.DMA,
        ],
    )
    def k(table_hbm, idx_hbm, out_hbm, idx_v, rows0, rows1, sem0, sem1):
        wid = lax.axis_index("s") * nc + lax.axis_index("c")
        out0 = wid * rows_per_w

        pltpu.sync_copy(idx_hbm.at[pl.ds(out0, rows_per_w)], idx_v)

        def fire(g, rows_v, sem):
            for j in range(_K):
                pltpu.async_copy(
                    table_hbm.at[idx_v.at[pl.ds(g * _CHUNK + j * _LANES, _LANES)]],
                    rows_v.at[pl.ds(j * _LANES, _LANES)],
                    sem,
                )

        def drain_wait(sem):
            for j in range(_K):
                pltpu.make_async_copy(
                    table_hbm.at[idx_v.at[pl.ds(j * _LANES, _LANES)]],
                    rows0.at[pl.ds(j * _LANES, _LANES)],
                    sem,
                ).wait()

        def store(g, rows_v):
            pltpu.sync_copy(
                rows_v,
                out_hbm.at[pl.ds(out0 + g * _CHUNK, _CHUNK)],
            )

        fire(0, rows0, sem0)

        def body(p, carry):
            g = 2 * p
            fire(g + 1, rows1, sem1)
            drain_wait(sem0)
            store(g, rows0)
            fire(g + 2, rows0, sem0)
            drain_wait(sem1)
            store(g + 1, rows1)
            return carry

        lax.fori_loop(0, n_pairs - 1, body, 0)

        g_last = n_chunks - 2
        fire(g_last + 1, rows1, sem1)
        drain_wait(sem0)
        store(g_last, rows0)
        drain_wait(sem1)
        store(g_last + 1, rows1)

    return k


def kernel(inputs, embed_table):
    b, s = inputs.shape
    v, d = embed_table.shape
    total = b * s
    idx_flat = inputs.reshape(total).astype(jnp.int32)
    tab_p = jnp.pad(embed_table, ((0, 0), (0, _PADW - d)))
    out = _make_gather(total, d)(tab_p, idx_flat)
    return out[:, :d].reshape(b, s, d)
